# Initial kernel scaffold; baseline (speedup 1.0000x reference)
import jax
import jax.numpy as jnp
from jax.experimental import pallas as pl


def kernel(x, pos, edge_index, W1, b1, A1, W2, b2, A2, Wu, bu):
    raise NotImplementedError("write your pallas kernel here")


# speedup vs baseline: 2.8280x; 2.8280x over previous
import functools

import jax
import jax.numpy as jnp
from jax import lax
from jax.experimental import pallas as pl
from jax.experimental.pallas import tpu as pltpu
from jax.experimental.pallas import tpu_sc as plsc

N_NODES = 10000
N_EDGES = 320000
D = 128
T = 144
N_P = 10240
NC = 2
NS = 16
NW = NC * NS
EPW = N_EDGES // NW
CH = 80
NCH = EPW // CH
RPT = N_P // NS


def _swish(v):
    return v * jax.nn.sigmoid(v)


def _sc_mesh():
    return plsc.VectorSubcoreMesh(
        core_axis_name="c", subcore_axis_name="s", num_cores=NC, num_subcores=NS
    )


def _sc_gather(xp, srcdst):
    @functools.partial(
        pl.kernel,
        out_type=[
            jax.ShapeDtypeStruct((N_EDGES, T), jnp.float32),
            jax.ShapeDtypeStruct((N_EDGES, T), jnp.float32),
        ],
        mesh=_sc_mesh(),
        scratch_types=[
            pltpu.VMEM((NCH, CH), jnp.int32),
            pltpu.VMEM((NCH, CH), jnp.int32),
            pltpu.VMEM((CH, T), jnp.float32),
            pltpu.VMEM((CH, T), jnp.float32),
            pltpu.SemaphoreType.DMA,
            pltpu.SemaphoreType.DMA,
        ],
        compiler_params=pltpu.CompilerParams(use_tc_tiling_on_sc=False),
    )
    def k(xp_hbm, idx_hbm, xs_hbm, xd_hbm, si_v, di_v, rs_v, rd_v, sem_s, sem_d):
        c = lax.axis_index("c")
        s = lax.axis_index("s")
        wid = c * NS + s
        base = wid * EPW
        pltpu.sync_copy(idx_hbm.at[0, wid], si_v)
        pltpu.sync_copy(idx_hbm.at[1, wid], di_v)

        def body(j, carry):
            off = base + j * CH
            cp_s = pltpu.async_copy(xp_hbm.at[si_v.at[j]], rs_v, sem_s)
            cp_d = pltpu.async_copy(xp_hbm.at[di_v.at[j]], rd_v, sem_d)
            cp_s.wait()
            cp_d.wait()
            pltpu.sync_copy(rs_v, xs_hbm.at[pl.ds(off, CH), :])
            pltpu.sync_copy(rd_v, xd_hbm.at[pl.ds(off, CH), :])
            return carry

        lax.fori_loop(0, NCH, body, 0)

    return k(xp, srcdst)


def _sc_scatter(m_aug, dst_r, zeros_hbm):
    @functools.partial(
        pl.kernel,
        out_type=jax.ShapeDtypeStruct((NC, N_P, T), jnp.float32),
        mesh=_sc_mesh(),
        scratch_types=[
            pltpu.VMEM((NCH, CH), jnp.int32),
            pltpu.VMEM((CH, T), jnp.float32),
            pltpu.VMEM_SHARED((N_P, T), jnp.float32),
        ],
        compiler_params=pltpu.CompilerParams(use_tc_tiling_on_sc=False),
    )
    def k(m_hbm, dst_hbm, z_hbm, out_hbm, di_v, rows_v, acc):
        c = lax.axis_index("c")
        s = lax.axis_index("s")
        wid = c * NS + s
        base = wid * EPW
        pltpu.sync_copy(z_hbm.at[pl.ds(s * RPT, RPT), :], acc.at[pl.ds(s * RPT, RPT), :])
        pltpu.sync_copy(dst_hbm.at[wid], di_v)
        plsc.subcore_barrier()

        def body(j, carry):
            off = base + j * CH
            pltpu.sync_copy(m_hbm.at[pl.ds(off, CH), :], rows_v)
            pltpu.sync_copy(rows_v, acc.at[di_v.at[j]], add=True)
            return carry

        lax.fori_loop(0, NCH, body, 0)
        plsc.subcore_barrier()
        pltpu.sync_copy(
            acc.at[pl.ds(s * RPT, RPT), :], out_hbm.at[c, pl.ds(s * RPT, RPT), :]
        )

    return k(m_aug, dst_r, zeros_hbm)


def _tc_messages(xs, xd, W1s, W1d, w1e, A1, b1, W2, A2, b2):
    B = 2000
    grid = N_EDGES // B

    def body(xs_ref, xd_ref, W1s_ref, W1d_ref, w1e_ref, A1_ref, b1_ref,
             W2_ref, A2_ref, b2_ref, out_ref):
        xs_b = xs_ref[:, :D]
        xd_b = xd_ref[:, :D]
        rel = xs_ref[:, D:D + 3] - xd_ref[:, D:D + 3]
        dist = jnp.sqrt(jnp.sum(rel * rel, axis=1, keepdims=True) + 1e-8)
        shv = rel / dist

        h = jnp.dot(xs_b, W1s_ref[:, :], preferred_element_type=jnp.float32)
        h += jnp.dot(xd_b, W1d_ref[:, :], preferred_element_type=jnp.float32)
        h += dist * w1e_ref[:, :]
        h += A1_ref[0:1, :] + b1_ref[:, :]
        for i in range(3):
            h += shv[:, i:i + 1] * A1_ref[i + 1:i + 2, :]
        h = _swish(h)

        h2 = jnp.dot(h, W2_ref[:, :], preferred_element_type=jnp.float32)
        h2 += A2_ref[0:1, :] + b2_ref[:, :]
        for i in range(3):
            h2 += shv[:, i:i + 1] * A2_ref[i + 1:i + 2, :]
        h2 = _swish(h2)

        ones = jnp.ones((B, 1), jnp.float32)
        pad = jnp.zeros((B, T - D - 5), jnp.float32)
        out_ref[:, :] = jnp.concatenate([h2, ones, shv, ones, pad], axis=1)

    wspec = lambda shape: pl.BlockSpec(shape, lambda i: (0, 0))
    return pl.pallas_call(
        body,
        grid=(grid,),
        in_specs=[
            pl.BlockSpec((B, T), lambda i: (i, 0)),
            pl.BlockSpec((B, T), lambda i: (i, 0)),
            wspec((D, D)), wspec((D, D)), wspec((1, D)), wspec((4, D)),
            wspec((1, D)), wspec((D, D)), wspec((4, D)), wspec((1, D)),
        ],
        out_specs=pl.BlockSpec((B, T), lambda i: (i, 0)),
        out_shape=jax.ShapeDtypeStruct((N_EDGES, T), jnp.float32),
    )(xs, xd, W1s, W1d, w1e, A1, b1, W2, A2, b2)


def _tc_update(xp, partials, Wux, Wua, Wun, bu):
    B = 2048
    grid = N_P // B

    def body(xp_ref, p0_ref, p1_ref, Wux_ref, Wua_ref, Wun_ref, bu_ref, out_ref):
        x_b = xp_ref[:, :D]
        p0 = p0_ref[0]
        p1 = p1_ref[0]
        agg = p0[:, :D] + p1[:, :D]
        sh_sum = p0[:, D:D + 4] + p1[:, D:D + 4]
        deg = jnp.maximum(p0[:, D + 4:D + 5] + p1[:, D + 4:D + 5], 1.0)
        node_attr = sh_sum / deg

        u = jnp.dot(x_b, Wux_ref[:, :], preferred_element_type=jnp.float32)
        u += jnp.dot(agg, Wua_ref[:, :], preferred_element_type=jnp.float32)
        for i in range(4):
            u += node_attr[:, i:i + 1] * Wun_ref[i:i + 1, :]
        u += bu_ref[:, :]
        out_ref[:, :] = x_b + _swish(u)

    wspec = lambda shape: pl.BlockSpec(shape, lambda i: (0, 0))
    return pl.pallas_call(
        body,
        grid=(grid,),
        in_specs=[
            pl.BlockSpec((B, T), lambda i: (i, 0)),
            pl.BlockSpec((1, B, T), lambda i: (0, i, 0)),
            pl.BlockSpec((1, B, T), lambda i: (1, i, 0)),
            wspec((D, D)), wspec((D, D)), wspec((4, D)), wspec((1, D)),
        ],
        out_specs=pl.BlockSpec((B, D), lambda i: (i, 0)),
        out_shape=jax.ShapeDtypeStruct((N_P, D), jnp.float32),
    )(xp, partials, partials, Wux, Wua, Wun, bu)


def kernel(x, pos, edge_index, W1, b1, A1, W2, b2, A2, Wu, bu):
    xp = jnp.zeros((N_P, T), jnp.float32)
    xp = xp.at[:N_NODES, :D].set(x).at[:N_NODES, D:D + 3].set(pos)
    srcdst = edge_index.reshape(2, NW, NCH, CH)
    zeros = jnp.zeros((N_P, T), jnp.float32)

    xs, xd = _sc_gather(xp, srcdst)
    m_aug = _tc_messages(
        xs, xd,
        W1[:D], W1[D:2 * D], W1[2 * D:2 * D + 1],
        A1, b1.reshape(1, D), W2, A2, b2.reshape(1, D),
    )
    partials = _sc_scatter(m_aug, srcdst[1], zeros)
    x_new_p = _tc_update(
        xp, partials,
        Wu[:D], Wu[D:2 * D], Wu[2 * D:2 * D + 4], bu.reshape(1, D),
    )
    return x_new_p[:N_NODES]


# speedup vs baseline: 4.5835x; 1.6208x over previous
import functools

import jax
import jax.numpy as jnp
from jax import lax
from jax.experimental import pallas as pl
from jax.experimental.pallas import tpu as pltpu
from jax.experimental.pallas import tpu_sc as plsc

N_NODES = 10000
N_EDGES = 320000
D = 128
T = 144
N_P = 10240
NC = 2
NS = 16
NW = NC * NS
EPW = N_EDGES // NW
CH = 80
NCH = EPW // CH
PPC = CH // 16
RPT = N_P // NS
NPOS = N_EDGES // 16


def _swish(v):
    return v * jax.nn.sigmoid(v)


def _sc_mesh():
    return plsc.VectorSubcoreMesh(
        core_axis_name="c", subcore_axis_name="s", num_cores=NC, num_subcores=NS
    )


def _tc_prep(xpad, W1s, W1d, c1):
    B = 2048

    def body(x_ref, W1s_ref, W1d_ref, c1_ref, ys_ref, yd_ref):
        x_b = x_ref[:, :]
        ys_ref[:, :] = (
            jnp.dot(x_b, W1s_ref[:, :], preferred_element_type=jnp.float32)
            + c1_ref[:, :]
        )
        yd_ref[:, :] = jnp.dot(x_b, W1d_ref[:, :], preferred_element_type=jnp.float32)

    wspec = lambda shape: pl.BlockSpec(shape, lambda i: (0, 0))
    return pl.pallas_call(
        body,
        grid=(N_P // B,),
        in_specs=[
            pl.BlockSpec((B, D), lambda i: (i, 0)),
            wspec((D, D)), wspec((D, D)), wspec((1, D)),
        ],
        out_specs=[
            pl.BlockSpec((B, D), lambda i: (i, 0)),
            pl.BlockSpec((B, D), lambda i: (i, 0)),
        ],
        out_shape=[
            jax.ShapeDtypeStruct((N_P, D), jnp.float32),
            jax.ShapeDtypeStruct((N_P, D), jnp.float32),
        ],
    )(xpad, W1s, W1d, c1)


GCH = 128
NT = N_EDGES // GCH
GPP = GCH // 16


def _sc_gather(ys, yd, posf, src_f, dst_f):
    @functools.partial(
        pl.kernel,
        out_type=[
            jax.ShapeDtypeStruct((N_EDGES, D), jnp.float32),
            jax.ShapeDtypeStruct((8, N_EDGES), jnp.float32),
        ],
        mesh=_sc_mesh(),
        scratch_types=[
            pltpu.VMEM((GCH,), jnp.int32),
            pltpu.VMEM((GCH,), jnp.int32),
            pltpu.VMEM((GCH, D), jnp.float32),
            pltpu.VMEM((8, GCH), jnp.float32),
            pltpu.VMEM((4 * N_P,), jnp.float32),
            pltpu.SemaphoreType.DMA,
            pltpu.SemaphoreType.DMA,
        ],
        compiler_params=pltpu.CompilerParams(needs_layout_passes=False),
    )
    def k(ys_hbm, yd_hbm, posf_hbm, src_hbm, dst_hbm, hpre_hbm, posp_hbm,
          si_v, di_v, buf, pbuf, posv, sem_a, sem_b):
        c = lax.axis_index("c")
        s = lax.axis_index("s")
        wid = c * NS + s
        pltpu.sync_copy(posf_hbm, posv)
        lane16 = lax.iota(jnp.int32, 16)

        def body(j, carry):
            t = j * NW + wid

            @pl.when(t < NT)
            def _():
                pltpu.sync_copy(src_hbm.at[pl.ds(t * GCH, GCH)], si_v)
                pltpu.sync_copy(dst_hbm.at[pl.ds(t * GCH, GCH)], di_v)
                pltpu.async_copy(ys_hbm.at[si_v], buf, sem_a).wait()
                pltpu.async_copy(yd_hbm.at[di_v], buf, sem_b, add=True).wait()
                pltpu.sync_copy(buf, hpre_hbm.at[pl.ds(t * GCH, GCH), :])
                for kk in range(GPP):
                    sl = pl.ds(kk * 16, 16)
                    s4 = si_v[sl] * 4
                    d4 = di_v[sl] * 4
                    cols = lane16 + kk * 16
                    for comp in range(3):
                        psc = plsc.load_gather(posv, [s4 + comp])
                        pdc = plsc.load_gather(posv, [d4 + comp])
                        plsc.store_scatter(
                            pbuf, [jnp.full((16,), comp, jnp.int32), cols], psc)
                        plsc.store_scatter(
                            pbuf, [jnp.full((16,), comp + 3, jnp.int32), cols], pdc)
                pltpu.sync_copy(pbuf, posp_hbm.at[:, pl.ds(t * GCH, GCH)])

            return carry

        lax.fori_loop(0, (NT + NW - 1) // NW, body, 0)

    return k(ys, yd, posf, src_f, dst_f)


def _sc_scatter(m_aug, dst_r, zeros_hbm):
    @functools.partial(
        pl.kernel,
        out_type=jax.ShapeDtypeStruct((NC, N_P, T), jnp.float32),
        mesh=_sc_mesh(),
        scratch_types=[
            pltpu.VMEM((NCH, CH), jnp.int32),
            pltpu.VMEM((CH, T), jnp.float32),
            pltpu.VMEM_SHARED((N_P, T), jnp.float32),
        ],
        compiler_params=pltpu.CompilerParams(use_tc_tiling_on_sc=False),
    )
    def k(m_hbm, dst_hbm, z_hbm, out_hbm, di_v, rows_v, acc):
        c = lax.axis_index("c")
        s = lax.axis_index("s")
        wid = c * NS + s
        base = wid * EPW
        pltpu.sync_copy(z_hbm.at[pl.ds(s * RPT, RPT), :], acc.at[pl.ds(s * RPT, RPT), :])
        pltpu.sync_copy(dst_hbm.at[wid], di_v)
        plsc.subcore_barrier()

        def body(j, carry):
            off = base + j * CH
            pltpu.sync_copy(m_hbm.at[pl.ds(off, CH), :], rows_v)
            pltpu.sync_copy(rows_v, acc.at[di_v.at[j]], add=True)
            return carry

        lax.fori_loop(0, NCH, body, 0)
        plsc.subcore_barrier()
        pltpu.sync_copy(
            acc.at[pl.ds(s * RPT, RPT), :], out_hbm.at[c, pl.ds(s * RPT, RPT), :]
        )

    return k(m_aug, dst_r, zeros_hbm)


def _tc_messages(hpre, pospT, W2, G1, G2, SSel):
    B = 2560
    dims = (((0,), (0,)), ((), ()))

    def body(hpre_ref, posp_ref, W2_ref, G1_ref, G2_ref, SSel_ref, out_ref):
        P = posp_ref[:, :]
        rel = P[0:3, :] - P[3:6, :]
        dist = jnp.sqrt(jnp.sum(rel * rel, axis=0, keepdims=True) + 1e-8)
        shv = rel / dist
        Q = jnp.concatenate(
            [dist, shv, jnp.ones((1, B), jnp.float32),
             jnp.zeros((3, B), jnp.float32)], axis=0)

        t1 = lax.dot_general(Q, G1_ref[:, :], dims,
                             preferred_element_type=jnp.float32)
        h = _swish(hpre_ref[:, :] + t1)
        h2 = jnp.dot(h, W2_ref[:, :], preferred_element_type=jnp.float32)
        h2 += lax.dot_general(Q, G2_ref[:, :], dims,
                              preferred_element_type=jnp.float32)
        m = _swish(h2)
        sh16 = lax.dot_general(Q, SSel_ref[:, :], dims,
                               preferred_element_type=jnp.float32)
        out_ref[:, :] = jnp.concatenate([m, sh16], axis=1)

    wspec = lambda shape: pl.BlockSpec(shape, lambda i: (0, 0))
    return pl.pallas_call(
        body,
        grid=(N_EDGES // B,),
        in_specs=[
            pl.BlockSpec((B, D), lambda i: (i, 0)),
            pl.BlockSpec((8, B), lambda i: (0, i)),
            wspec((D, D)), wspec((8, D)), wspec((8, D)), wspec((8, 16)),
        ],
        out_specs=pl.BlockSpec((B, T), lambda i: (i, 0)),
        out_shape=jax.ShapeDtypeStruct((N_EDGES, T), jnp.float32),
    )(hpre, pospT, W2, G1, G2, SSel)


def _tc_update(xpad, partials, Wux, Wua, Wun, bu):
    B = 2048

    def body(x_ref, p0_ref, p1_ref, Wux_ref, Wua_ref, Wun_ref, bu_ref, out_ref):
        x_b = x_ref[:, :]
        p0 = p0_ref[0]
        p1 = p1_ref[0]
        agg = p0[:, :D] + p1[:, :D]
        sh_sum = p0[:, D:D + 4] + p1[:, D:D + 4]
        deg = jnp.maximum(p0[:, D + 4:D + 5] + p1[:, D + 4:D + 5], 1.0)
        node_attr = sh_sum / deg

        u = jnp.dot(x_b, Wux_ref[:, :], preferred_element_type=jnp.float32)
        u += jnp.dot(agg, Wua_ref[:, :], preferred_element_type=jnp.float32)
        for i in range(4):
            u += node_attr[:, i:i + 1] * Wun_ref[i:i + 1, :]
        u += bu_ref[:, :]
        out_ref[:, :] = x_b + _swish(u)

    wspec = lambda shape: pl.BlockSpec(shape, lambda i: (0, 0))
    return pl.pallas_call(
        body,
        grid=(N_P // B,),
        in_specs=[
            pl.BlockSpec((B, D), lambda i: (i, 0)),
            pl.BlockSpec((1, B, T), lambda i: (0, i, 0)),
            pl.BlockSpec((1, B, T), lambda i: (1, i, 0)),
            wspec((D, D)), wspec((D, D)), wspec((4, D)), wspec((1, D)),
        ],
        out_specs=pl.BlockSpec((B, D), lambda i: (i, 0)),
        out_shape=jax.ShapeDtypeStruct((N_P, D), jnp.float32),
    )(xpad, partials, partials, Wux, Wua, Wun, bu)


def kernel(x, pos, edge_index, W1, b1, A1, W2, b2, A2, Wu, bu):
    xpad = jnp.zeros((N_P, D), jnp.float32).at[:N_NODES].set(x)
    posf = jnp.zeros((N_P, 4), jnp.float32).at[:N_NODES, :3].set(pos).reshape(-1)
    srcdst = edge_index.reshape(2, NW, NCH, CH)
    zeros = jnp.zeros((N_P, T), jnp.float32)

    c1 = (b1 + A1[0]).reshape(1, D)
    G1 = jnp.zeros((8, D), jnp.float32).at[0].set(W1[2 * D]).at[1:4].set(A1[1:4])
    G2 = jnp.zeros((8, D), jnp.float32).at[1:4].set(A2[1:4]).at[4].set(b2 + A2[0])
    SSel = (jnp.zeros((8, 16), jnp.float32)
            .at[4, 0].set(1.0).at[1, 1].set(1.0).at[2, 2].set(1.0)
            .at[3, 3].set(1.0).at[4, 4].set(1.0))

    ys, yd = _tc_prep(xpad, W1[:D], W1[D:2 * D], c1)
    hpre, pospT = _sc_gather(ys, yd, posf, edge_index[0], edge_index[1])
    m_aug = _tc_messages(hpre, pospT, W2, G1, G2, SSel)
    partials = _sc_scatter(m_aug, srcdst[1], zeros)
    x_new_p = _tc_update(
        xpad, partials, Wu[:D], Wu[D:2 * D], Wu[2 * D:2 * D + 4], bu.reshape(1, D)
    )
    return x_new_p[:N_NODES]


# speedup vs baseline: 5.1000x; 1.1127x over previous
import functools

import jax
import jax.numpy as jnp
from jax import lax
from jax.experimental import pallas as pl
from jax.experimental.pallas import tpu as pltpu
from jax.experimental.pallas import tpu_sc as plsc

N_NODES = 10000
N_EDGES = 320000
D = 128
T = 144
N_P = 10240
NC = 2
NS = 16
NW = NC * NS
EPW = N_EDGES // NW
CH = 80
NCH = EPW // CH
PPC = CH // 16
RPT = N_P // NS
NPOS = N_EDGES // 16


def _swish(v):
    return v * jax.nn.sigmoid(v)


def _sc_mesh():
    return plsc.VectorSubcoreMesh(
        core_axis_name="c", subcore_axis_name="s", num_cores=NC, num_subcores=NS
    )


def _tc_prep(xpad, W1s, W1d, c1):
    B = 2048

    def body(x_ref, W1s_ref, W1d_ref, c1_ref, ys_ref, yd_ref):
        x_b = x_ref[:, :]
        ys_ref[:, :] = (
            jnp.dot(x_b, W1s_ref[:, :], preferred_element_type=jnp.float32)
            + c1_ref[:, :]
        )
        yd_ref[:, :] = jnp.dot(x_b, W1d_ref[:, :], preferred_element_type=jnp.float32)

    wspec = lambda shape: pl.BlockSpec(shape, lambda i: (0, 0))
    return pl.pallas_call(
        body,
        grid=(N_P // B,),
        in_specs=[
            pl.BlockSpec((B, D), lambda i: (i, 0)),
            wspec((D, D)), wspec((D, D)), wspec((1, D)),
        ],
        out_specs=[
            pl.BlockSpec((B, D), lambda i: (i, 0)),
            pl.BlockSpec((B, D), lambda i: (i, 0)),
        ],
        out_shape=[
            jax.ShapeDtypeStruct((N_P, D), jnp.float32),
            jax.ShapeDtypeStruct((N_P, D), jnp.float32),
        ],
    )(xpad, W1s, W1d, c1)


GCH = 128
NT = N_EDGES // GCH
GPP = GCH // 16


def _sc_gather(ys, yd, posf, src_f, dst_f):
    e_loc = src_f.shape[0]
    nt = e_loc // GCH

    @functools.partial(
        pl.kernel,
        out_type=[
            jax.ShapeDtypeStruct((e_loc, D), jnp.float32),
            jax.ShapeDtypeStruct((8, e_loc), jnp.float32),
        ],
        mesh=_sc_mesh(),
        scratch_types=[
            pltpu.VMEM((GCH,), jnp.int32),
            pltpu.VMEM((GCH,), jnp.int32),
            pltpu.VMEM((GCH, D), jnp.float32),
            pltpu.VMEM((8, GCH), jnp.float32),
            pltpu.VMEM((4 * N_P,), jnp.float32),
            pltpu.SemaphoreType.DMA,
            pltpu.SemaphoreType.DMA,
        ],
        compiler_params=pltpu.CompilerParams(needs_layout_passes=False),
    )
    def k(ys_hbm, yd_hbm, posf_hbm, src_hbm, dst_hbm, hpre_hbm, posp_hbm,
          si_v, di_v, buf, pbuf, posv, sem_a, sem_b):
        c = lax.axis_index("c")
        s = lax.axis_index("s")
        wid = c * NS + s
        pltpu.sync_copy(posf_hbm, posv)
        lane16 = lax.iota(jnp.int32, 16)

        def body(j, carry):
            t = j * NW + wid

            @pl.when(t < nt)
            def _():
                pltpu.sync_copy(src_hbm.at[pl.ds(t * GCH, GCH)], si_v)
                pltpu.sync_copy(dst_hbm.at[pl.ds(t * GCH, GCH)], di_v)
                pltpu.async_copy(ys_hbm.at[si_v], buf, sem_a).wait()
                pltpu.async_copy(yd_hbm.at[di_v], buf, sem_b, add=True).wait()
                pltpu.sync_copy(buf, hpre_hbm.at[pl.ds(t * GCH, GCH), :])
                for kk in range(GPP):
                    sl = pl.ds(kk * 16, 16)
                    s4 = si_v[sl] * 4
                    d4 = di_v[sl] * 4
                    cols = lane16 + kk * 16
                    for comp in range(3):
                        psc = plsc.load_gather(posv, [s4 + comp])
                        pdc = plsc.load_gather(posv, [d4 + comp])
                        plsc.store_scatter(
                            pbuf, [jnp.full((16,), comp, jnp.int32), cols], psc)
                        plsc.store_scatter(
                            pbuf, [jnp.full((16,), comp + 3, jnp.int32), cols], pdc)
                pltpu.sync_copy(pbuf, posp_hbm.at[:, pl.ds(t * GCH, GCH)])

            return carry

        lax.fori_loop(0, (nt + NW - 1) // NW, body, 0)

    return k(ys, yd, posf, src_f, dst_f)


def _sc_scatter(m_aug, dst_r, zeros_hbm):
    nch_l = dst_r.shape[1]
    ch_l = dst_r.shape[2]
    epw_l = nch_l * ch_l

    @functools.partial(
        pl.kernel,
        out_type=jax.ShapeDtypeStruct((NC, N_P, T), jnp.float32),
        mesh=_sc_mesh(),
        scratch_types=[
            pltpu.VMEM((nch_l, ch_l), jnp.int32),
            pltpu.VMEM((ch_l, T), jnp.float32),
            pltpu.VMEM_SHARED((N_P, T), jnp.float32),
        ],
        compiler_params=pltpu.CompilerParams(use_tc_tiling_on_sc=False),
    )
    def k(m_hbm, dst_hbm, z_hbm, out_hbm, di_v, rows_v, acc):
        c = lax.axis_index("c")
        s = lax.axis_index("s")
        wid = c * NS + s
        base = wid * epw_l
        pltpu.sync_copy(z_hbm.at[pl.ds(s * RPT, RPT), :], acc.at[pl.ds(s * RPT, RPT), :])
        pltpu.sync_copy(dst_hbm.at[wid], di_v)
        plsc.subcore_barrier()

        def body(j, carry):
            off = base + j * ch_l
            pltpu.sync_copy(m_hbm.at[pl.ds(off, ch_l), :], rows_v)
            pltpu.sync_copy(rows_v, acc.at[di_v.at[j]], add=True)
            return carry

        lax.fori_loop(0, nch_l, body, 0)
        plsc.subcore_barrier()
        pltpu.sync_copy(
            acc.at[pl.ds(s * RPT, RPT), :], out_hbm.at[c, pl.ds(s * RPT, RPT), :]
        )

    return k(m_aug, dst_r, zeros_hbm)


def _tc_messages(hpre, pospT, W2, G1, G2, SSel):
    e_loc = hpre.shape[0]
    B = 1280 if e_loc % 2560 else 2560
    dims = (((0,), (0,)), ((), ()))

    def body(hpre_ref, posp_ref, W2_ref, G1_ref, G2_ref, SSel_ref, out_ref):
        P = posp_ref[:, :]
        rel = P[0:3, :] - P[3:6, :]
        dist = jnp.sqrt(jnp.sum(rel * rel, axis=0, keepdims=True) + 1e-8)
        shv = rel / dist
        Q = jnp.concatenate(
            [dist, shv, jnp.ones((1, B), jnp.float32),
             jnp.zeros((3, B), jnp.float32)], axis=0)

        t1 = lax.dot_general(Q, G1_ref[:, :], dims,
                             preferred_element_type=jnp.float32)
        h = _swish(hpre_ref[:, :] + t1)
        h2 = jnp.dot(h, W2_ref[:, :], preferred_element_type=jnp.float32)
        h2 += lax.dot_general(Q, G2_ref[:, :], dims,
                              preferred_element_type=jnp.float32)
        m = _swish(h2)
        sh16 = lax.dot_general(Q, SSel_ref[:, :], dims,
                               preferred_element_type=jnp.float32)
        out_ref[:, :] = jnp.concatenate([m, sh16], axis=1)

    wspec = lambda shape: pl.BlockSpec(shape, lambda i: (0, 0))
    return pl.pallas_call(
        body,
        grid=(e_loc // B,),
        in_specs=[
            pl.BlockSpec((B, D), lambda i: (i, 0)),
            pl.BlockSpec((8, B), lambda i: (0, i)),
            wspec((D, D)), wspec((8, D)), wspec((8, D)), wspec((8, 16)),
        ],
        out_specs=pl.BlockSpec((B, T), lambda i: (i, 0)),
        out_shape=jax.ShapeDtypeStruct((e_loc, T), jnp.float32),
    )(hpre, pospT, W2, G1, G2, SSel)


def _tc_update(xpad, partials_list, Wux, Wua, Wun, bu):
    B = 2048
    n_p = NC * len(partials_list)

    def body(x_ref, *refs):
        p_refs = refs[:n_p]
        Wux_ref, Wua_ref, Wun_ref, bu_ref, out_ref = refs[n_p:]
        x_b = x_ref[:, :]
        ptot = p_refs[0][0]
        for pr in p_refs[1:]:
            ptot = ptot + pr[0]
        agg = ptot[:, :D]
        sh_sum = ptot[:, D:D + 4]
        deg = jnp.maximum(ptot[:, D + 4:D + 5], 1.0)
        node_attr = sh_sum / deg

        u = jnp.dot(x_b, Wux_ref[:, :], preferred_element_type=jnp.float32)
        u += jnp.dot(agg, Wua_ref[:, :], preferred_element_type=jnp.float32)
        for i in range(4):
            u += node_attr[:, i:i + 1] * Wun_ref[i:i + 1, :]
        u += bu_ref[:, :]
        out_ref[:, :] = x_b + _swish(u)

    wspec = lambda shape: pl.BlockSpec(shape, lambda i: (0, 0))
    p_args = []
    p_specs = []
    for p in partials_list:
        for cc in range(NC):
            p_args.append(p)
            p_specs.append(pl.BlockSpec((1, B, T), lambda i, _c=cc: (_c, i, 0)))
    return pl.pallas_call(
        body,
        grid=(N_P // B,),
        in_specs=[pl.BlockSpec((B, D), lambda i: (i, 0))] + p_specs + [
            wspec((D, D)), wspec((D, D)), wspec((4, D)), wspec((1, D)),
        ],
        out_specs=pl.BlockSpec((B, D), lambda i: (i, 0)),
        out_shape=jax.ShapeDtypeStruct((N_P, D), jnp.float32),
    )(xpad, *p_args, Wux, Wua, Wun, bu)


K_SPLIT = 2


def kernel(x, pos, edge_index, W1, b1, A1, W2, b2, A2, Wu, bu):
    xpad = jnp.zeros((N_P, D), jnp.float32).at[:N_NODES].set(x)
    posf = jnp.zeros((N_P, 4), jnp.float32).at[:N_NODES, :3].set(pos).reshape(-1)
    zeros = jnp.zeros((N_P, T), jnp.float32)

    c1 = (b1 + A1[0]).reshape(1, D)
    G1 = jnp.zeros((8, D), jnp.float32).at[0].set(W1[2 * D]).at[1:4].set(A1[1:4])
    G2 = jnp.zeros((8, D), jnp.float32).at[1:4].set(A2[1:4]).at[4].set(b2 + A2[0])
    SSel = (jnp.zeros((8, 16), jnp.float32)
            .at[4, 0].set(1.0).at[1, 1].set(1.0).at[2, 2].set(1.0)
            .at[3, 3].set(1.0).at[4, 4].set(1.0))

    ys, yd = _tc_prep(xpad, W1[:D], W1[D:2 * D], c1)

    e_sl = N_EDGES // K_SPLIT
    ch_l = 40 if K_SPLIT > 1 else CH
    partials_list = []
    for kk in range(K_SPLIT):
        sl = slice(kk * e_sl, (kk + 1) * e_sl)
        hpre, pospT = _sc_gather(ys, yd, posf, edge_index[0, sl], edge_index[1, sl])
        m_aug = _tc_messages(hpre, pospT, W2, G1, G2, SSel)
        dst_r = edge_index[1, sl].reshape(NW, e_sl // (NW * ch_l), ch_l)
        partials_list.append(_sc_scatter(m_aug, dst_r, zeros))

    x_new_p = _tc_update(
        xpad, partials_list, Wu[:D], Wu[D:2 * D], Wu[2 * D:2 * D + 4],
        bu.reshape(1, D)
    )
    return x_new_p[:N_NODES]


# speedup vs baseline: 5.2728x; 1.0339x over previous
import functools

import jax
import jax.numpy as jnp
from jax import lax
from jax.experimental import pallas as pl
from jax.experimental.pallas import tpu as pltpu
from jax.experimental.pallas import tpu_sc as plsc

N_NODES = 10000
N_EDGES = 320000
D = 128
T = 144
N_P = 10240
NC = 2
NS = 16
NW = NC * NS
EPW = N_EDGES // NW
CH = 80
NCH = EPW // CH
PPC = CH // 16
RPT = N_P // NS
NPOS = N_EDGES // 16


def _swish(v):
    return v * jax.nn.sigmoid(v)


def _sc_mesh():
    return plsc.VectorSubcoreMesh(
        core_axis_name="c", subcore_axis_name="s", num_cores=NC, num_subcores=NS
    )


def _tc_prep(xpad, W1s, W1d, c1):
    B = 2048

    def body(x_ref, W1s_ref, W1d_ref, c1_ref, ys_ref, yd_ref):
        x_b = x_ref[:, :]
        ys_ref[:, :] = (
            jnp.dot(x_b, W1s_ref[:, :], preferred_element_type=jnp.float32)
            + c1_ref[:, :]
        )
        yd_ref[:, :] = jnp.dot(x_b, W1d_ref[:, :], preferred_element_type=jnp.float32)

    wspec = lambda shape: pl.BlockSpec(shape, lambda i: (0, 0))
    return pl.pallas_call(
        body,
        grid=(N_P // B,),
        in_specs=[
            pl.BlockSpec((B, D), lambda i: (i, 0)),
            wspec((D, D)), wspec((D, D)), wspec((1, D)),
        ],
        out_specs=[
            pl.BlockSpec((B, D), lambda i: (i, 0)),
            pl.BlockSpec((B, D), lambda i: (i, 0)),
        ],
        out_shape=[
            jax.ShapeDtypeStruct((N_P, D), jnp.float32),
            jax.ShapeDtypeStruct((N_P, D), jnp.float32),
        ],
    )(xpad, W1s, W1d, c1)


GCH = 128
NT = N_EDGES // GCH
GPP = GCH // 16


def _sc_gather(ys, yd, posf, src_f, dst_f):
    e_loc = src_f.shape[0]
    nt = e_loc // GCH

    @functools.partial(
        pl.kernel,
        out_type=[
            jax.ShapeDtypeStruct((e_loc, D), jnp.float32),
            jax.ShapeDtypeStruct((8, e_loc), jnp.float32),
        ],
        mesh=_sc_mesh(),
        scratch_types=[
            pltpu.VMEM((GCH,), jnp.int32),
            pltpu.VMEM((GCH,), jnp.int32),
            pltpu.VMEM((GCH, D), jnp.float32),
            pltpu.VMEM((8, GCH), jnp.float32),
            pltpu.VMEM((4 * N_P,), jnp.float32),
            pltpu.SemaphoreType.DMA,
            pltpu.SemaphoreType.DMA,
        ],
        compiler_params=pltpu.CompilerParams(needs_layout_passes=False),
    )
    def k(ys_hbm, yd_hbm, posf_hbm, src_hbm, dst_hbm, hpre_hbm, posp_hbm,
          si_v, di_v, buf, pbuf, posv, sem_a, sem_b):
        c = lax.axis_index("c")
        s = lax.axis_index("s")
        wid = c * NS + s
        pltpu.sync_copy(posf_hbm, posv)
        lane16 = lax.iota(jnp.int32, 16)

        def body(j, carry):
            t = j * NW + wid

            @pl.when(t < nt)
            def _():
                pltpu.sync_copy(src_hbm.at[pl.ds(t * GCH, GCH)], si_v)
                pltpu.sync_copy(dst_hbm.at[pl.ds(t * GCH, GCH)], di_v)
                pltpu.async_copy(ys_hbm.at[si_v], buf, sem_a).wait()
                pltpu.async_copy(yd_hbm.at[di_v], buf, sem_b, add=True).wait()
                pltpu.sync_copy(buf, hpre_hbm.at[pl.ds(t * GCH, GCH), :])
                for kk in range(GPP):
                    sl = pl.ds(kk * 16, 16)
                    s4 = si_v[sl] * 4
                    d4 = di_v[sl] * 4
                    cols = lane16 + kk * 16
                    for comp in range(3):
                        psc = plsc.load_gather(posv, [s4 + comp])
                        pdc = plsc.load_gather(posv, [d4 + comp])
                        plsc.store_scatter(
                            pbuf, [jnp.full((16,), comp, jnp.int32), cols], psc)
                        plsc.store_scatter(
                            pbuf, [jnp.full((16,), comp + 3, jnp.int32), cols], pdc)
                pltpu.sync_copy(pbuf, posp_hbm.at[:, pl.ds(t * GCH, GCH)])

            return carry

        lax.fori_loop(0, (nt + NW - 1) // NW, body, 0)

    return k(ys, yd, posf, src_f, dst_f)


def _sc_scatter(m_aug, dst_r, zeros_hbm):
    nch_l = dst_r.shape[1]
    ch_l = dst_r.shape[2]
    epw_l = nch_l * ch_l

    n_pair = nch_l // 2
    tail = nch_l % 2

    @functools.partial(
        pl.kernel,
        out_type=jax.ShapeDtypeStruct((NC, N_P, T), jnp.float32),
        mesh=_sc_mesh(),
        scratch_types=[
            pltpu.VMEM((nch_l, ch_l), jnp.int32),
            pltpu.VMEM((2 * ch_l, T), jnp.float32),
            pltpu.VMEM_SHARED((N_P, T), jnp.float32),
        ],
        compiler_params=pltpu.CompilerParams(use_tc_tiling_on_sc=False),
    )
    def k(m_hbm, dst_hbm, z_hbm, out_hbm, di_v, rows_v, acc):
        c = lax.axis_index("c")
        s = lax.axis_index("s")
        wid = c * NS + s
        base = wid * epw_l
        pltpu.sync_copy(z_hbm.at[pl.ds(s * RPT, RPT), :], acc.at[pl.ds(s * RPT, RPT), :])
        pltpu.sync_copy(dst_hbm.at[wid], di_v)
        plsc.subcore_barrier()

        def body(j, carry):
            off = base + j * 2 * ch_l
            pltpu.sync_copy(m_hbm.at[pl.ds(off, 2 * ch_l), :], rows_v)
            pltpu.sync_copy(rows_v.at[pl.ds(0, ch_l), :],
                            acc.at[di_v.at[2 * j]], add=True)
            pltpu.sync_copy(rows_v.at[pl.ds(ch_l, ch_l), :],
                            acc.at[di_v.at[2 * j + 1]], add=True)
            return carry

        lax.fori_loop(0, n_pair, body, 0)
        if tail:
            off = base + n_pair * 2 * ch_l
            pltpu.sync_copy(m_hbm.at[pl.ds(off, ch_l), :],
                            rows_v.at[pl.ds(0, ch_l), :])
            pltpu.sync_copy(rows_v.at[pl.ds(0, ch_l), :],
                            acc.at[di_v.at[nch_l - 1]], add=True)
        plsc.subcore_barrier()
        pltpu.sync_copy(
            acc.at[pl.ds(s * RPT, RPT), :], out_hbm.at[c, pl.ds(s * RPT, RPT), :]
        )

    return k(m_aug, dst_r, zeros_hbm)


def _tc_messages(hpre, pospT, W2, G1, G2, SSel):
    e_loc = hpre.shape[0]
    B = 1280 if e_loc % 2560 else 2560
    dims = (((0,), (0,)), ((), ()))

    def body(hpre_ref, posp_ref, W2_ref, G1_ref, G2_ref, SSel_ref, out_ref):
        P = posp_ref[:, :]
        rel = P[0:3, :] - P[3:6, :]
        dist = jnp.sqrt(jnp.sum(rel * rel, axis=0, keepdims=True) + 1e-8)
        shv = rel / dist
        Q = jnp.concatenate(
            [dist, shv, jnp.ones((1, B), jnp.float32),
             jnp.zeros((3, B), jnp.float32)], axis=0)

        t1 = lax.dot_general(Q, G1_ref[:, :], dims,
                             preferred_element_type=jnp.float32)
        h = _swish(hpre_ref[:, :] + t1)
        h2 = jnp.dot(h, W2_ref[:, :], preferred_element_type=jnp.float32)
        h2 += lax.dot_general(Q, G2_ref[:, :], dims,
                              preferred_element_type=jnp.float32)
        m = _swish(h2)
        sh16 = lax.dot_general(Q, SSel_ref[:, :], dims,
                               preferred_element_type=jnp.float32)
        out_ref[:, :] = jnp.concatenate([m, sh16], axis=1)

    wspec = lambda shape: pl.BlockSpec(shape, lambda i: (0, 0))
    return pl.pallas_call(
        body,
        grid=(e_loc // B,),
        in_specs=[
            pl.BlockSpec((B, D), lambda i: (i, 0)),
            pl.BlockSpec((8, B), lambda i: (0, i)),
            wspec((D, D)), wspec((8, D)), wspec((8, D)), wspec((8, 16)),
        ],
        out_specs=pl.BlockSpec((B, T), lambda i: (i, 0)),
        out_shape=jax.ShapeDtypeStruct((e_loc, T), jnp.float32),
    )(hpre, pospT, W2, G1, G2, SSel)


def _tc_update(xpad, partials_list, Wux, Wua, Wun, bu):
    B = 2048
    n_p = NC * len(partials_list)

    def body(x_ref, *refs):
        p_refs = refs[:n_p]
        Wux_ref, Wua_ref, Wun_ref, bu_ref, out_ref = refs[n_p:]
        x_b = x_ref[:, :]
        ptot = p_refs[0][0]
        for pr in p_refs[1:]:
            ptot = ptot + pr[0]
        agg = ptot[:, :D]
        sh_sum = ptot[:, D:D + 4]
        deg = jnp.maximum(ptot[:, D + 4:D + 5], 1.0)
        node_attr = sh_sum / deg

        u = jnp.dot(x_b, Wux_ref[:, :], preferred_element_type=jnp.float32)
        u += jnp.dot(agg, Wua_ref[:, :], preferred_element_type=jnp.float32)
        for i in range(4):
            u += node_attr[:, i:i + 1] * Wun_ref[i:i + 1, :]
        u += bu_ref[:, :]
        out_ref[:, :] = x_b + _swish(u)

    wspec = lambda shape: pl.BlockSpec(shape, lambda i: (0, 0))
    p_args = []
    p_specs = []
    for p in partials_list:
        for cc in range(NC):
            p_args.append(p)
            p_specs.append(pl.BlockSpec((1, B, T), lambda i, _c=cc: (_c, i, 0)))
    return pl.pallas_call(
        body,
        grid=(N_P // B,),
        in_specs=[pl.BlockSpec((B, D), lambda i: (i, 0))] + p_specs + [
            wspec((D, D)), wspec((D, D)), wspec((4, D)), wspec((1, D)),
        ],
        out_specs=pl.BlockSpec((B, D), lambda i: (i, 0)),
        out_shape=jax.ShapeDtypeStruct((N_P, D), jnp.float32),
    )(xpad, *p_args, Wux, Wua, Wun, bu)


K_SPLIT = 2


def kernel(x, pos, edge_index, W1, b1, A1, W2, b2, A2, Wu, bu):
    xpad = jnp.zeros((N_P, D), jnp.float32).at[:N_NODES].set(x)
    posf = jnp.zeros((N_P, 4), jnp.float32).at[:N_NODES, :3].set(pos).reshape(-1)
    zeros = jnp.zeros((N_P, T), jnp.float32)

    c1 = (b1 + A1[0]).reshape(1, D)
    G1 = jnp.zeros((8, D), jnp.float32).at[0].set(W1[2 * D]).at[1:4].set(A1[1:4])
    G2 = jnp.zeros((8, D), jnp.float32).at[1:4].set(A2[1:4]).at[4].set(b2 + A2[0])
    SSel = (jnp.zeros((8, 16), jnp.float32)
            .at[4, 0].set(1.0).at[1, 1].set(1.0).at[2, 2].set(1.0)
            .at[3, 3].set(1.0).at[4, 4].set(1.0))

    ys, yd = _tc_prep(xpad, W1[:D], W1[D:2 * D], c1)

    e_sl = N_EDGES // K_SPLIT
    ch_l = 40 if K_SPLIT > 1 else CH
    partials_list = []
    for kk in range(K_SPLIT):
        sl = slice(kk * e_sl, (kk + 1) * e_sl)
        hpre, pospT = _sc_gather(ys, yd, posf, edge_index[0, sl], edge_index[1, sl])
        m_aug = _tc_messages(hpre, pospT, W2, G1, G2, SSel)
        dst_r = edge_index[1, sl].reshape(NW, e_sl // (NW * ch_l), ch_l)
        partials_list.append(_sc_scatter(m_aug, dst_r, zeros))

    x_new_p = _tc_update(
        xpad, partials_list, Wu[:D], Wu[D:2 * D], Wu[2 * D:2 * D + 4],
        bu.reshape(1, D)
    )
    return x_new_p[:N_NODES]


# speedup vs baseline: 5.6942x; 1.0799x over previous
import functools

import jax
import jax.numpy as jnp
from jax import lax
from jax.experimental import pallas as pl
from jax.experimental.pallas import tpu as pltpu
from jax.experimental.pallas import tpu_sc as plsc

N_NODES = 10000
N_EDGES = 320000
D = 128
T = 144
N_P = 10240
NC = 2
NS = 16
NW = NC * NS
EPW = N_EDGES // NW
CH = 80
NCH = EPW // CH
PPC = CH // 16
RPT = N_P // NS
NPOS = N_EDGES // 16


def _swish(v):
    return v * jax.nn.sigmoid(v)


def _sc_mesh():
    return plsc.VectorSubcoreMesh(
        core_axis_name="c", subcore_axis_name="s", num_cores=NC, num_subcores=NS
    )


def _tc_prep(xpad, W1s, W1d, c1):
    B = 2048

    def body(x_ref, W1s_ref, W1d_ref, c1_ref, ys_ref, yd_ref):
        x_b = x_ref[:, :]
        ys_ref[:, :] = (
            jnp.dot(x_b, W1s_ref[:, :], preferred_element_type=jnp.float32)
            + c1_ref[:, :]
        )
        yd_ref[:, :] = jnp.dot(x_b, W1d_ref[:, :], preferred_element_type=jnp.float32)

    wspec = lambda shape: pl.BlockSpec(shape, lambda i: (0, 0))
    return pl.pallas_call(
        body,
        grid=(N_P // B,),
        in_specs=[
            pl.BlockSpec((B, D), lambda i: (i, 0)),
            wspec((D, D)), wspec((D, D)), wspec((1, D)),
        ],
        out_specs=[
            pl.BlockSpec((B, D), lambda i: (i, 0)),
            pl.BlockSpec((B, D), lambda i: (i, 0)),
        ],
        out_shape=[
            jax.ShapeDtypeStruct((N_P, D), jnp.float32),
            jax.ShapeDtypeStruct((N_P, D), jnp.float32),
        ],
    )(xpad, W1s, W1d, c1)


GCH = 128
NT = N_EDGES // GCH
GPP = GCH // 16


def _sc_gather(ys, yd, posf, src_f, dst_f):
    e_loc = src_f.shape[0]
    nt = e_loc // GCH

    max_n = (nt + NW - 1) // NW

    @functools.partial(
        pl.kernel,
        out_type=[
            jax.ShapeDtypeStruct((e_loc, D), jnp.float32),
            jax.ShapeDtypeStruct((8, e_loc), jnp.float32),
        ],
        mesh=_sc_mesh(),
        scratch_types=(
            [pltpu.VMEM((GCH,), jnp.int32)] * 6
            + [pltpu.VMEM((GCH, D), jnp.float32)] * 3
            + [pltpu.VMEM((8, GCH), jnp.float32),
               pltpu.VMEM((4 * N_P,), jnp.float32)]
            + [pltpu.SemaphoreType.DMA] * 9
        ),
        compiler_params=pltpu.CompilerParams(needs_layout_passes=False),
    )
    def k(ys_hbm, yd_hbm, posf_hbm, src_hbm, dst_hbm, hpre_hbm, posp_hbm,
          si0, si1, si2, di0, di1, di2, b0, b1, b2, pbuf, posv,
          mi0, mi1, mi2, mg0, mg1, mg2, mh0, mh1, mh2):
        si = (si0, si1, si2)
        di = (di0, di1, di2)
        bufs = (b0, b1, b2)
        smi = (mi0, mi1, mi2)
        smg = (mg0, mg1, mg2)
        smh = (mh0, mh1, mh2)
        c = lax.axis_index("c")
        s = lax.axis_index("s")
        wid = c * NS + s
        pltpu.sync_copy(posf_hbm, posv)
        lane16 = lax.iota(jnp.int32, 16)

        @pl.when(wid < nt)
        def _():
            t0 = wid
            pltpu.async_copy(src_hbm.at[pl.ds(t0 * GCH, GCH)], si[0], smi[0])
            pltpu.async_copy(dst_hbm.at[pl.ds(t0 * GCH, GCH)], di[0], smi[0])

        def group(g, carry):
            for b in range(3):
                k3 = g * 3 + b
                bh = (b + 2) % 3
                bc = (b + 1) % 3
                t_g = k3 * NW + wid
                t_h = (k3 - 1) * NW + wid
                t_c = (k3 - 2) * NW + wid
                t_n = (k3 + 1) * NW + wid

                @pl.when(t_g < nt)
                def _(b=b, t_g=t_g):
                    pltpu.make_async_copy(
                        src_hbm.at[pl.ds(0, GCH)], si[b], smi[b]).wait()
                    pltpu.make_async_copy(
                        dst_hbm.at[pl.ds(0, GCH)], di[b], smi[b]).wait()
                    pltpu.async_copy(ys_hbm.at[si[b]], bufs[b], smg[b])

                @pl.when(jnp.logical_and(k3 >= 1, t_h < nt))
                def _(bh=bh):
                    pltpu.make_async_copy(
                        ys_hbm.at[pl.ds(0, GCH)], bufs[bh], smg[bh]).wait()
                    pltpu.async_copy(yd_hbm.at[di[bh]], bufs[bh], smh[bh], add=True)

                @pl.when(jnp.logical_and(k3 >= 2, t_c < nt))
                def _(bc=bc, t_c=t_c):
                    pltpu.make_async_copy(
                        yd_hbm.at[pl.ds(0, GCH)], bufs[bc], smh[bc]).wait()
                    pltpu.sync_copy(bufs[bc], hpre_hbm.at[pl.ds(t_c * GCH, GCH), :])
                    for kk in range(GPP):
                        sl = pl.ds(kk * 16, 16)
                        s4 = si[bc][sl] * 4
                        d4 = di[bc][sl] * 4
                        cols = lane16 + kk * 16
                        for comp in range(3):
                            psc = plsc.load_gather(posv, [s4 + comp])
                            pdc = plsc.load_gather(posv, [d4 + comp])
                            plsc.store_scatter(
                                pbuf, [jnp.full((16,), comp, jnp.int32), cols], psc)
                            plsc.store_scatter(
                                pbuf, [jnp.full((16,), comp + 3, jnp.int32), cols],
                                pdc)
                    pltpu.sync_copy(pbuf, posp_hbm.at[:, pl.ds(t_c * GCH, GCH)])

                @pl.when(t_n < nt)
                def _(bc=bc, t_n=t_n):
                    pltpu.async_copy(
                        src_hbm.at[pl.ds(t_n * GCH, GCH)], si[bc], smi[bc])
                    pltpu.async_copy(
                        dst_hbm.at[pl.ds(t_n * GCH, GCH)], di[bc], smi[bc])

            return carry

        lax.fori_loop(0, (max_n + 2 + 2) // 3, group, 0)

    return k(ys, yd, posf, src_f, dst_f)


def _sc_scatter(m_aug, dst_r, zeros_hbm):
    nch_l = dst_r.shape[1]
    ch_l = dst_r.shape[2]
    epw_l = nch_l * ch_l

    n_pair = nch_l // 2
    tail = nch_l % 2

    @functools.partial(
        pl.kernel,
        out_type=jax.ShapeDtypeStruct((NC, N_P, T), jnp.float32),
        mesh=_sc_mesh(),
        scratch_types=[
            pltpu.VMEM((nch_l, ch_l), jnp.int32),
            pltpu.VMEM((2 * ch_l, T), jnp.float32),
            pltpu.VMEM_SHARED((N_P, T), jnp.float32),
        ],
        compiler_params=pltpu.CompilerParams(use_tc_tiling_on_sc=False),
    )
    def k(m_hbm, dst_hbm, z_hbm, out_hbm, di_v, rows_v, acc):
        c = lax.axis_index("c")
        s = lax.axis_index("s")
        wid = c * NS + s
        base = wid * epw_l
        pltpu.sync_copy(z_hbm.at[pl.ds(s * RPT, RPT), :], acc.at[pl.ds(s * RPT, RPT), :])
        pltpu.sync_copy(dst_hbm.at[wid], di_v)
        plsc.subcore_barrier()

        def body(j, carry):
            off = base + j * 2 * ch_l
            pltpu.sync_copy(m_hbm.at[pl.ds(off, 2 * ch_l), :], rows_v)
            pltpu.sync_copy(rows_v.at[pl.ds(0, ch_l), :],
                            acc.at[di_v.at[2 * j]], add=True)
            pltpu.sync_copy(rows_v.at[pl.ds(ch_l, ch_l), :],
                            acc.at[di_v.at[2 * j + 1]], add=True)
            return carry

        lax.fori_loop(0, n_pair, body, 0)
        if tail:
            off = base + n_pair * 2 * ch_l
            pltpu.sync_copy(m_hbm.at[pl.ds(off, ch_l), :],
                            rows_v.at[pl.ds(0, ch_l), :])
            pltpu.sync_copy(rows_v.at[pl.ds(0, ch_l), :],
                            acc.at[di_v.at[nch_l - 1]], add=True)
        plsc.subcore_barrier()
        pltpu.sync_copy(
            acc.at[pl.ds(s * RPT, RPT), :], out_hbm.at[c, pl.ds(s * RPT, RPT), :]
        )

    return k(m_aug, dst_r, zeros_hbm)


def _tc_messages(hpre, pospT, W2, G1, G2, SSel):
    e_loc = hpre.shape[0]
    B = 1280 if e_loc % 2560 else 2560
    dims = (((0,), (0,)), ((), ()))

    def body(hpre_ref, posp_ref, W2_ref, G1_ref, G2_ref, SSel_ref, out_ref):
        P = posp_ref[:, :]
        rel = P[0:3, :] - P[3:6, :]
        dist = jnp.sqrt(jnp.sum(rel * rel, axis=0, keepdims=True) + 1e-8)
        shv = rel / dist
        Q = jnp.concatenate(
            [dist, shv, jnp.ones((1, B), jnp.float32),
             jnp.zeros((3, B), jnp.float32)], axis=0)

        t1 = lax.dot_general(Q, G1_ref[:, :], dims,
                             preferred_element_type=jnp.float32)
        h = _swish(hpre_ref[:, :] + t1)
        h2 = jnp.dot(h, W2_ref[:, :], preferred_element_type=jnp.float32)
        h2 += lax.dot_general(Q, G2_ref[:, :], dims,
                              preferred_element_type=jnp.float32)
        m = _swish(h2)
        sh16 = lax.dot_general(Q, SSel_ref[:, :], dims,
                               preferred_element_type=jnp.float32)
        out_ref[:, :] = jnp.concatenate([m, sh16], axis=1)

    wspec = lambda shape: pl.BlockSpec(shape, lambda i: (0, 0))
    return pl.pallas_call(
        body,
        grid=(e_loc // B,),
        in_specs=[
            pl.BlockSpec((B, D), lambda i: (i, 0)),
            pl.BlockSpec((8, B), lambda i: (0, i)),
            wspec((D, D)), wspec((8, D)), wspec((8, D)), wspec((8, 16)),
        ],
        out_specs=pl.BlockSpec((B, T), lambda i: (i, 0)),
        out_shape=jax.ShapeDtypeStruct((e_loc, T), jnp.float32),
    )(hpre, pospT, W2, G1, G2, SSel)


def _tc_update(xpad, partials_list, Wux, Wua, Wun, bu):
    B = 2048
    n_p = NC * len(partials_list)

    def body(x_ref, *refs):
        p_refs = refs[:n_p]
        Wux_ref, Wua_ref, Wun_ref, bu_ref, out_ref = refs[n_p:]
        x_b = x_ref[:, :]
        ptot = p_refs[0][0]
        for pr in p_refs[1:]:
            ptot = ptot + pr[0]
        agg = ptot[:, :D]
        sh_sum = ptot[:, D:D + 4]
        deg = jnp.maximum(ptot[:, D + 4:D + 5], 1.0)
        node_attr = sh_sum / deg

        u = jnp.dot(x_b, Wux_ref[:, :], preferred_element_type=jnp.float32)
        u += jnp.dot(agg, Wua_ref[:, :], preferred_element_type=jnp.float32)
        for i in range(4):
            u += node_attr[:, i:i + 1] * Wun_ref[i:i + 1, :]
        u += bu_ref[:, :]
        out_ref[:, :] = x_b + _swish(u)

    wspec = lambda shape: pl.BlockSpec(shape, lambda i: (0, 0))
    p_args = []
    p_specs = []
    for p in partials_list:
        for cc in range(NC):
            p_args.append(p)
            p_specs.append(pl.BlockSpec((1, B, T), lambda i, _c=cc: (_c, i, 0)))
    return pl.pallas_call(
        body,
        grid=(N_P // B,),
        in_specs=[pl.BlockSpec((B, D), lambda i: (i, 0))] + p_specs + [
            wspec((D, D)), wspec((D, D)), wspec((4, D)), wspec((1, D)),
        ],
        out_specs=pl.BlockSpec((B, D), lambda i: (i, 0)),
        out_shape=jax.ShapeDtypeStruct((N_P, D), jnp.float32),
    )(xpad, *p_args, Wux, Wua, Wun, bu)


K_SPLIT = 2


def kernel(x, pos, edge_index, W1, b1, A1, W2, b2, A2, Wu, bu):
    xpad = jnp.zeros((N_P, D), jnp.float32).at[:N_NODES].set(x)
    posf = jnp.zeros((N_P, 4), jnp.float32).at[:N_NODES, :3].set(pos).reshape(-1)
    zeros = jnp.zeros((N_P, T), jnp.float32)

    c1 = (b1 + A1[0]).reshape(1, D)
    G1 = jnp.zeros((8, D), jnp.float32).at[0].set(W1[2 * D]).at[1:4].set(A1[1:4])
    G2 = jnp.zeros((8, D), jnp.float32).at[1:4].set(A2[1:4]).at[4].set(b2 + A2[0])
    SSel = (jnp.zeros((8, 16), jnp.float32)
            .at[4, 0].set(1.0).at[1, 1].set(1.0).at[2, 2].set(1.0)
            .at[3, 3].set(1.0).at[4, 4].set(1.0))

    ys, yd = _tc_prep(xpad, W1[:D], W1[D:2 * D], c1)

    e_sl = N_EDGES // K_SPLIT
    ch_l = 40 if K_SPLIT > 1 else CH
    partials_list = []
    for kk in range(K_SPLIT):
        sl = slice(kk * e_sl, (kk + 1) * e_sl)
        hpre, pospT = _sc_gather(ys, yd, posf, edge_index[0, sl], edge_index[1, sl])
        m_aug = _tc_messages(hpre, pospT, W2, G1, G2, SSel)
        dst_r = edge_index[1, sl].reshape(NW, e_sl // (NW * ch_l), ch_l)
        partials_list.append(_sc_scatter(m_aug, dst_r, zeros))

    x_new_p = _tc_update(
        xpad, partials_list, Wu[:D], Wu[D:2 * D], Wu[2 * D:2 * D + 4],
        bu.reshape(1, D)
    )
    return x_new_p[:N_NODES]


# speedup vs baseline: 5.8782x; 1.0323x over previous
import functools

import jax
import jax.numpy as jnp
from jax import lax
from jax.experimental import pallas as pl
from jax.experimental.pallas import tpu as pltpu
from jax.experimental.pallas import tpu_sc as plsc

N_NODES = 10000
N_EDGES = 320000
D = 128
T = 144
N_P = 10240
NC = 2
NS = 16
NW = NC * NS
EPW = N_EDGES // NW
CH = 80
NCH = EPW // CH
PPC = CH // 16
RPT = N_P // NS
NPOS = N_EDGES // 16


def _swish(v):
    return v * jax.nn.sigmoid(v)


def _sc_mesh():
    return plsc.VectorSubcoreMesh(
        core_axis_name="c", subcore_axis_name="s", num_cores=NC, num_subcores=NS
    )


def _tc_prep(xpad, W1s, W1d, c1):
    B = 2048

    def body(x_ref, W1s_ref, W1d_ref, c1_ref, ys_ref, yd_ref):
        x_b = x_ref[:, :]
        ys_ref[:, :] = (
            jnp.dot(x_b, W1s_ref[:, :], preferred_element_type=jnp.float32)
            + c1_ref[:, :]
        )
        yd_ref[:, :] = jnp.dot(x_b, W1d_ref[:, :], preferred_element_type=jnp.float32)

    wspec = lambda shape: pl.BlockSpec(shape, lambda i: (0, 0))
    return pl.pallas_call(
        body,
        grid=(N_P // B,),
        in_specs=[
            pl.BlockSpec((B, D), lambda i: (i, 0)),
            wspec((D, D)), wspec((D, D)), wspec((1, D)),
        ],
        out_specs=[
            pl.BlockSpec((B, D), lambda i: (i, 0)),
            pl.BlockSpec((B, D), lambda i: (i, 0)),
        ],
        out_shape=[
            jax.ShapeDtypeStruct((N_P, D), jnp.float32),
            jax.ShapeDtypeStruct((N_P, D), jnp.float32),
        ],
    )(xpad, W1s, W1d, c1)


GCH = 128
NT = N_EDGES // GCH
GPP = GCH // 16


def _sc_gather(ys, yd, posf, src_f, dst_f):
    e_loc = src_f.shape[0]
    nt = e_loc // GCH

    max_n = (nt + NW - 1) // NW

    @functools.partial(
        pl.kernel,
        out_type=[
            jax.ShapeDtypeStruct((e_loc, D), jnp.float32),
            jax.ShapeDtypeStruct((8, e_loc), jnp.float32),
        ],
        mesh=_sc_mesh(),
        scratch_types=(
            [pltpu.VMEM((GCH,), jnp.int32)] * 6
            + [pltpu.VMEM((GCH, D), jnp.float32)] * 3
            + [pltpu.VMEM((8, GCH), jnp.float32),
               pltpu.VMEM((4 * N_P,), jnp.float32)]
            + [pltpu.SemaphoreType.DMA] * 9
        ),
        compiler_params=pltpu.CompilerParams(needs_layout_passes=False),
    )
    def k(ys_hbm, yd_hbm, posf_hbm, src_hbm, dst_hbm, hpre_hbm, posp_hbm,
          si0, si1, si2, di0, di1, di2, b0, b1, b2, pbuf, posv,
          mi0, mi1, mi2, mg0, mg1, mg2, mh0, mh1, mh2):
        si = (si0, si1, si2)
        di = (di0, di1, di2)
        bufs = (b0, b1, b2)
        smi = (mi0, mi1, mi2)
        smg = (mg0, mg1, mg2)
        smh = (mh0, mh1, mh2)
        c = lax.axis_index("c")
        s = lax.axis_index("s")
        wid = c * NS + s
        pltpu.sync_copy(posf_hbm, posv)
        lane16 = lax.iota(jnp.int32, 16)

        @pl.when(wid < nt)
        def _():
            t0 = wid
            pltpu.async_copy(src_hbm.at[pl.ds(t0 * GCH, GCH)], si[0], smi[0])
            pltpu.async_copy(dst_hbm.at[pl.ds(t0 * GCH, GCH)], di[0], smi[0])

        def group(g, carry):
            for b in range(3):
                k3 = g * 3 + b
                bh = (b + 2) % 3
                bc = (b + 1) % 3
                t_g = k3 * NW + wid
                t_h = (k3 - 1) * NW + wid
                t_c = (k3 - 2) * NW + wid
                t_n = (k3 + 1) * NW + wid

                @pl.when(t_g < nt)
                def _(b=b, t_g=t_g):
                    pltpu.make_async_copy(
                        src_hbm.at[pl.ds(0, GCH)], si[b], smi[b]).wait()
                    pltpu.make_async_copy(
                        dst_hbm.at[pl.ds(0, GCH)], di[b], smi[b]).wait()
                    pltpu.async_copy(ys_hbm.at[si[b]], bufs[b], smg[b])

                @pl.when(jnp.logical_and(k3 >= 1, t_h < nt))
                def _(bh=bh):
                    pltpu.make_async_copy(
                        ys_hbm.at[pl.ds(0, GCH)], bufs[bh], smg[bh]).wait()
                    pltpu.async_copy(yd_hbm.at[di[bh]], bufs[bh], smh[bh], add=True)

                @pl.when(jnp.logical_and(k3 >= 2, t_c < nt))
                def _(bc=bc, t_c=t_c):
                    pltpu.make_async_copy(
                        yd_hbm.at[pl.ds(0, GCH)], bufs[bc], smh[bc]).wait()
                    pltpu.sync_copy(bufs[bc], hpre_hbm.at[pl.ds(t_c * GCH, GCH), :])
                    for kk in range(GPP):
                        sl = pl.ds(kk * 16, 16)
                        s4 = si[bc][sl] * 4
                        d4 = di[bc][sl] * 4
                        cols = lane16 + kk * 16
                        for comp in range(3):
                            psc = plsc.load_gather(posv, [s4 + comp])
                            pdc = plsc.load_gather(posv, [d4 + comp])
                            plsc.store_scatter(
                                pbuf, [jnp.full((16,), comp, jnp.int32), cols], psc)
                            plsc.store_scatter(
                                pbuf, [jnp.full((16,), comp + 3, jnp.int32), cols],
                                pdc)
                    pltpu.sync_copy(pbuf, posp_hbm.at[:, pl.ds(t_c * GCH, GCH)])

                @pl.when(t_n < nt)
                def _(bc=bc, t_n=t_n):
                    pltpu.async_copy(
                        src_hbm.at[pl.ds(t_n * GCH, GCH)], si[bc], smi[bc])
                    pltpu.async_copy(
                        dst_hbm.at[pl.ds(t_n * GCH, GCH)], di[bc], smi[bc])

            return carry

        lax.fori_loop(0, (max_n + 2 + 2) // 3, group, 0)

    return k(ys, yd, posf, src_f, dst_f)


def _sc_scatter(m_aug, dst_r, zeros_hbm):
    nch_l = dst_r.shape[1]
    ch_l = dst_r.shape[2]
    epw_l = nch_l * ch_l

    n_pair = nch_l // 2
    tail = nch_l % 2

    @functools.partial(
        pl.kernel,
        out_type=jax.ShapeDtypeStruct((NC, N_P, T), jnp.float32),
        mesh=_sc_mesh(),
        scratch_types=[
            pltpu.VMEM((nch_l, ch_l), jnp.int32),
            pltpu.VMEM((2 * ch_l, T), jnp.float32),
            pltpu.VMEM((2 * ch_l, T), jnp.float32),
            pltpu.VMEM_SHARED((N_P, T), jnp.float32),
            pltpu.SemaphoreType.DMA,
            pltpu.SemaphoreType.DMA,
        ],
        compiler_params=pltpu.CompilerParams(use_tc_tiling_on_sc=False),
    )
    def k(m_hbm, dst_hbm, z_hbm, out_hbm, di_v, r0, r1, acc, sm0, sm1):
        rbs = (r0, r1)
        sms = (sm0, sm1)
        c = lax.axis_index("c")
        s = lax.axis_index("s")
        wid = c * NS + s
        base = wid * epw_l
        pltpu.sync_copy(z_hbm.at[pl.ds(s * RPT, RPT), :], acc.at[pl.ds(s * RPT, RPT), :])
        pltpu.sync_copy(dst_hbm.at[wid], di_v)
        plsc.subcore_barrier()

        pltpu.async_copy(m_hbm.at[pl.ds(base, 2 * ch_l), :], rbs[0], sms[0])

        def body(g, carry):
            for b in range(2):
                jj = g * 2 + b
                pltpu.make_async_copy(
                    m_hbm.at[pl.ds(0, 2 * ch_l), :], rbs[b], sms[b]).wait()

                @pl.when(jj + 1 < n_pair)
                def _(b=b, jj=jj):
                    off_n = base + (jj + 1) * 2 * ch_l
                    pltpu.async_copy(
                        m_hbm.at[pl.ds(off_n, 2 * ch_l), :], rbs[1 - b], sms[1 - b])

                pltpu.sync_copy(rbs[b].at[pl.ds(0, ch_l), :],
                                acc.at[di_v.at[2 * jj]], add=True)
                pltpu.sync_copy(rbs[b].at[pl.ds(ch_l, ch_l), :],
                                acc.at[di_v.at[2 * jj + 1]], add=True)
            return carry

        lax.fori_loop(0, n_pair // 2, body, 0)
        for jj in range(n_pair // 2 * 2, n_pair):
            b = jj % 2
            pltpu.make_async_copy(
                m_hbm.at[pl.ds(0, 2 * ch_l), :], rbs[b], sms[b]).wait()
            pltpu.sync_copy(rbs[b].at[pl.ds(0, ch_l), :],
                            acc.at[di_v.at[2 * jj]], add=True)
            pltpu.sync_copy(rbs[b].at[pl.ds(ch_l, ch_l), :],
                            acc.at[di_v.at[2 * jj + 1]], add=True)
        if tail:
            off = base + n_pair * 2 * ch_l
            pltpu.sync_copy(m_hbm.at[pl.ds(off, ch_l), :],
                            r0.at[pl.ds(0, ch_l), :])
            pltpu.sync_copy(r0.at[pl.ds(0, ch_l), :],
                            acc.at[di_v.at[nch_l - 1]], add=True)
        plsc.subcore_barrier()
        pltpu.sync_copy(
            acc.at[pl.ds(s * RPT, RPT), :], out_hbm.at[c, pl.ds(s * RPT, RPT), :]
        )

    return k(m_aug, dst_r, zeros_hbm)


def _tc_messages(hpre, pospT, W2, G1, G2, SSel):
    e_loc = hpre.shape[0]
    B = 1280 if e_loc % 2560 else 2560
    dims = (((0,), (0,)), ((), ()))

    def body(hpre_ref, posp_ref, W2_ref, G1_ref, G2_ref, SSel_ref, out_ref):
        P = posp_ref[:, :]
        rel = P[0:3, :] - P[3:6, :]
        dist = jnp.sqrt(jnp.sum(rel * rel, axis=0, keepdims=True) + 1e-8)
        shv = rel / dist
        Q = jnp.concatenate(
            [dist, shv, jnp.ones((1, B), jnp.float32),
             jnp.zeros((3, B), jnp.float32)], axis=0)

        t1 = lax.dot_general(Q, G1_ref[:, :], dims,
                             preferred_element_type=jnp.float32)
        h = _swish(hpre_ref[:, :] + t1)
        h2 = jnp.dot(h, W2_ref[:, :], preferred_element_type=jnp.float32)
        h2 += lax.dot_general(Q, G2_ref[:, :], dims,
                              preferred_element_type=jnp.float32)
        m = _swish(h2)
        sh16 = lax.dot_general(Q, SSel_ref[:, :], dims,
                               preferred_element_type=jnp.float32)
        out_ref[:, :] = jnp.concatenate([m, sh16], axis=1)

    wspec = lambda shape: pl.BlockSpec(shape, lambda i: (0, 0))
    return pl.pallas_call(
        body,
        grid=(e_loc // B,),
        in_specs=[
            pl.BlockSpec((B, D), lambda i: (i, 0)),
            pl.BlockSpec((8, B), lambda i: (0, i)),
            wspec((D, D)), wspec((8, D)), wspec((8, D)), wspec((8, 16)),
        ],
        out_specs=pl.BlockSpec((B, T), lambda i: (i, 0)),
        out_shape=jax.ShapeDtypeStruct((e_loc, T), jnp.float32),
    )(hpre, pospT, W2, G1, G2, SSel)


def _tc_update(xpad, partials_list, Wux, Wua, Wun, bu):
    B = 2048
    n_p = NC * len(partials_list)

    def body(x_ref, *refs):
        p_refs = refs[:n_p]
        Wux_ref, Wua_ref, Wun_ref, bu_ref, out_ref = refs[n_p:]
        x_b = x_ref[:, :]
        ptot = p_refs[0][0]
        for pr in p_refs[1:]:
            ptot = ptot + pr[0]
        agg = ptot[:, :D]
        sh_sum = ptot[:, D:D + 4]
        deg = jnp.maximum(ptot[:, D + 4:D + 5], 1.0)
        node_attr = sh_sum / deg

        u = jnp.dot(x_b, Wux_ref[:, :], preferred_element_type=jnp.float32)
        u += jnp.dot(agg, Wua_ref[:, :], preferred_element_type=jnp.float32)
        for i in range(4):
            u += node_attr[:, i:i + 1] * Wun_ref[i:i + 1, :]
        u += bu_ref[:, :]
        out_ref[:, :] = x_b + _swish(u)

    wspec = lambda shape: pl.BlockSpec(shape, lambda i: (0, 0))
    p_args = []
    p_specs = []
    for p in partials_list:
        for cc in range(NC):
            p_args.append(p)
            p_specs.append(pl.BlockSpec((1, B, T), lambda i, _c=cc: (_c, i, 0)))
    return pl.pallas_call(
        body,
        grid=(N_P // B,),
        in_specs=[pl.BlockSpec((B, D), lambda i: (i, 0))] + p_specs + [
            wspec((D, D)), wspec((D, D)), wspec((4, D)), wspec((1, D)),
        ],
        out_specs=pl.BlockSpec((B, D), lambda i: (i, 0)),
        out_shape=jax.ShapeDtypeStruct((N_P, D), jnp.float32),
    )(xpad, *p_args, Wux, Wua, Wun, bu)


K_SPLIT = 2


def kernel(x, pos, edge_index, W1, b1, A1, W2, b2, A2, Wu, bu):
    xpad = jnp.zeros((N_P, D), jnp.float32).at[:N_NODES].set(x)
    posf = jnp.zeros((N_P, 4), jnp.float32).at[:N_NODES, :3].set(pos).reshape(-1)
    zeros = jnp.zeros((N_P, T), jnp.float32)

    c1 = (b1 + A1[0]).reshape(1, D)
    G1 = jnp.zeros((8, D), jnp.float32).at[0].set(W1[2 * D]).at[1:4].set(A1[1:4])
    G2 = jnp.zeros((8, D), jnp.float32).at[1:4].set(A2[1:4]).at[4].set(b2 + A2[0])
    SSel = (jnp.zeros((8, 16), jnp.float32)
            .at[4, 0].set(1.0).at[1, 1].set(1.0).at[2, 2].set(1.0)
            .at[3, 3].set(1.0).at[4, 4].set(1.0))

    ys, yd = _tc_prep(xpad, W1[:D], W1[D:2 * D], c1)

    e_sl = N_EDGES // K_SPLIT
    ch_l = 40 if K_SPLIT > 1 else CH
    partials_list = []
    for kk in range(K_SPLIT):
        sl = slice(kk * e_sl, (kk + 1) * e_sl)
        hpre, pospT = _sc_gather(ys, yd, posf, edge_index[0, sl], edge_index[1, sl])
        m_aug = _tc_messages(hpre, pospT, W2, G1, G2, SSel)
        dst_r = edge_index[1, sl].reshape(NW, e_sl // (NW * ch_l), ch_l)
        partials_list.append(_sc_scatter(m_aug, dst_r, zeros))

    x_new_p = _tc_update(
        xpad, partials_list, Wu[:D], Wu[D:2 * D], Wu[2 * D:2 * D + 4],
        bu.reshape(1, D)
    )
    return x_new_p[:N_NODES]
